# baseline (speedup 1.0000x reference)
import jax
import jax.numpy as jnp
from jax import lax
from jax.experimental import pallas as pl
from jax.experimental.pallas import tpu as pltpu
from jax.experimental.pallas import tpu_sc as plsc

D = 64
L = 3
C = 1024
NC = 2
NS = 16
NW = NC * NS
LANES = 16
K = 256
DP = 128


def _sc_body(ids_hbm, lookup_hbm, sid_hbm, ind_hbm, out_hbm,
             ids_v, cidx_v, sidx_v, tmp_v, acc_v, out_v,
             sem_codes, sem_ind, sem_sid):
    n_total = ids_hbm.shape[0]
    per_w = n_total // NW
    n_chunks = per_w // K
    wid = lax.axis_index("s") * NC + lax.axis_index("c")

    def chunk_body(ci, carry):
        base = wid * per_w + ci * K
        pltpu.sync_copy(ids_hbm.at[pl.ds(base, K)], ids_v)
        ind_dma = pltpu.async_copy(ind_hbm.at[ids_v], acc_v, sem_ind)
        for c in range(K // LANES):
            v = ids_v[pl.ds(c * LANES, LANES)] * L
            for l in range(L):
                cidx_v[pl.ds(l * K + c * LANES, LANES)] = v + l
        pltpu.async_copy(lookup_hbm.at[cidx_v], sidx_v, sem_codes).wait()
        for l in range(L):
            off = jnp.int32(l * C + 1)
            for c in range(K // LANES):
                s = pl.ds(l * K + c * LANES, LANES)
                sidx_v[s] = sidx_v[s] + off
        pltpu.async_copy(sid_hbm.at[sidx_v], tmp_v, sem_sid).wait()
        ind_dma.wait()

        def add_body(i, carry2):
            for c in range(D // LANES):
                s = pl.ds(c * LANES, LANES)
                out_v[i, s] = (acc_v[i, s] + tmp_v[i, s]
                               + tmp_v[K + i, s] + tmp_v[2 * K + i, s])
            return carry2

        lax.fori_loop(0, K, add_body, 0)
        pltpu.sync_copy(out_v, out_hbm.at[pl.ds(base, K)])
        return carry

    lax.fori_loop(0, n_chunks, chunk_body, 0)


def _impl(ids, lookup_flat, sid_table, ind_table):
    n = ids.shape[0]
    mesh = plsc.VectorSubcoreMesh(core_axis_name="c", subcore_axis_name="s")
    fn = pl.kernel(
        _sc_body,
        out_type=jax.ShapeDtypeStruct((n, D), jnp.float32),
        mesh=mesh,
        compiler_params=pltpu.CompilerParams(use_tc_tiling_on_sc=False),
        scratch_types=[
            pltpu.VMEM((K,), jnp.int32),
            pltpu.VMEM((L * K,), jnp.int32),
            pltpu.VMEM((L * K,), jnp.int32),
            pltpu.VMEM((L * K, D), jnp.float32),
            pltpu.VMEM((K, DP), jnp.float32),
            pltpu.VMEM((K, D), jnp.float32),
            pltpu.SemaphoreType.DMA,
            pltpu.SemaphoreType.DMA,
            pltpu.SemaphoreType.DMA,
        ],
    )
    return fn(ids, lookup_flat, sid_table, ind_table)


def kernel(item_ids, lookup, codebook, sid_table, ind_table):
    b, t = item_ids.shape
    ids = item_ids.reshape(-1)
    lookup_flat = lookup.reshape(-1)
    ind_pad = jnp.pad(ind_table, ((0, 0), (0, DP - D)))
    out = _impl(ids, lookup_flat, sid_table, ind_pad)
    return out.reshape(b, t, D)


# speedup vs baseline: 1.0810x; 1.0810x over previous
import jax
import jax.numpy as jnp
from jax import lax
from jax.experimental import pallas as pl
from jax.experimental.pallas import tpu as pltpu
from jax.experimental.pallas import tpu_sc as plsc

D = 64
L = 3
C = 1024
NC = 2
NS = 16
NW = NC * NS
LANES = 16
K = 256
DP = 128


def _sc_body(ids_hbm, lookup_hbm, sid_hbm, ind_hbm, out_hbm,
             ids_v, cidx_v, sidx_v, tmp_v, acc_v,
             sem_codes, sem_ind, sem_sid):
    n_total = ids_hbm.shape[0]
    per_w = n_total // NW
    n_chunks = per_w // K
    wid = lax.axis_index("s") * NC + lax.axis_index("c")

    def chunk_body(ci, carry):
        base = wid * per_w + ci * K
        pltpu.sync_copy(ids_hbm.at[pl.ds(base, K)], ids_v)
        ind_dma = pltpu.async_copy(ind_hbm.at[ids_v], acc_v, sem_ind)
        for c in range(K // LANES):
            v = ids_v[pl.ds(c * LANES, LANES)] * L
            for l in range(L):
                cidx_v[pl.ds(l * K + c * LANES, LANES)] = v + l
        pltpu.async_copy(lookup_hbm.at[cidx_v], sidx_v, sem_codes).wait()
        for l in range(L):
            off = jnp.int32(l * C + 1)
            for c in range(K // LANES):
                s = pl.ds(l * K + c * LANES, LANES)
                sidx_v[s] = sidx_v[s] + off
        pltpu.async_copy(sid_hbm.at[sidx_v], tmp_v, sem_sid).wait()
        ind_dma.wait()

        def add_body(i, carry2):
            for c in range(D // LANES):
                s = pl.ds(c * LANES, LANES)
                acc_v[i, s] = (acc_v[i, s] + tmp_v[i, s]
                               + tmp_v[K + i, s] + tmp_v[2 * K + i, s])
            return carry2

        lax.fori_loop(0, K, add_body, 0)
        pltpu.sync_copy(acc_v, out_hbm.at[pl.ds(base, K)])
        return carry

    lax.fori_loop(0, n_chunks, chunk_body, 0)


def _impl(ids, lookup_flat, sid_table, ind_table):
    n = ids.shape[0]
    mesh = plsc.VectorSubcoreMesh(core_axis_name="c", subcore_axis_name="s")
    fn = pl.kernel(
        _sc_body,
        out_type=jax.ShapeDtypeStruct((n, D), jnp.float32),
        mesh=mesh,
        compiler_params=pltpu.CompilerParams(use_tc_tiling_on_sc=False),
        scratch_types=[
            pltpu.VMEM((K,), jnp.int32),
            pltpu.VMEM((L * K,), jnp.int32),
            pltpu.VMEM((L * K,), jnp.int32),
            pltpu.VMEM((L * K, D), jnp.float32),
            pltpu.VMEM((K, D), jnp.float32),
            pltpu.SemaphoreType.DMA,
            pltpu.SemaphoreType.DMA,
            pltpu.SemaphoreType.DMA,
        ],
    )
    return fn(ids, lookup_flat, sid_table, ind_table)


def kernel(item_ids, lookup, codebook, sid_table, ind_table):
    b, t = item_ids.shape
    ids = item_ids.reshape(-1)
    lookup_flat = lookup.reshape(-1)
    ind_lin = jax.lax.optimization_barrier(ind_table.reshape(-1))
    ind2 = ind_lin.reshape(ind_table.shape)
    out = _impl(ids, lookup_flat, sid_table, ind2)
    return out.reshape(b, t, D)


# speedup vs baseline: 1.0962x; 1.0140x over previous
import jax
import jax.numpy as jnp
from jax import lax
from jax.experimental import pallas as pl
from jax.experimental.pallas import tpu as pltpu
from jax.experimental.pallas import tpu_sc as plsc

D = 64
L = 3
C = 1024
NC = 2
NS = 16
NW = NC * NS
LANES = 16
K = 256
DP = 128


def _sc_body(ids_hbm, lookup_hbm, sid_hbm, ind_hbm, out_hbm,
             ids_v, cidx_v, iidx_v, sidx_v, tmp_v, acc_v,
             sem_codes, sem_ind, sem_sid):
    n_total = ids_hbm.shape[0]
    per_w = n_total // NW
    n_chunks = per_w // K
    wid = lax.axis_index("s") * NC + lax.axis_index("c")

    def chunk_body(ci, carry):
        base = wid * per_w + ci * K
        pltpu.sync_copy(ids_hbm.at[pl.ds(base, K)], ids_v)
        for c in range(K // LANES):
            v = ids_v[pl.ds(c * LANES, LANES)]
            iidx_v[pl.ds(c * LANES, LANES)] = v + v
            v = v * L
            for l in range(L):
                cidx_v[pl.ds(l * K + c * LANES, LANES)] = v + l
        codes_dma = pltpu.async_copy(lookup_hbm.at[cidx_v], sidx_v, sem_codes)
        ind_dma = pltpu.async_copy(ind_hbm.at[iidx_v], acc_v, sem_ind)
        codes_dma.wait()
        for l in range(L):
            off = jnp.int32(l * C + 1)
            for c in range(K // LANES):
                s = pl.ds(l * K + c * LANES, LANES)
                sidx_v[s] = sidx_v[s] + off
        pltpu.async_copy(sid_hbm.at[sidx_v], tmp_v, sem_sid).wait()
        ind_dma.wait()

        def add_body(i, carry2):
            for c in range(D // LANES):
                s = pl.ds(c * LANES, LANES)
                acc_v[i, s] = (acc_v[i, s] + tmp_v[i, s]
                               + tmp_v[K + i, s] + tmp_v[2 * K + i, s])
            return carry2

        lax.fori_loop(0, K, add_body, 0)
        pltpu.sync_copy(acc_v, out_hbm.at[pl.ds(base, K)])
        return carry

    lax.fori_loop(0, n_chunks, chunk_body, 0)


def _impl(ids, lookup_flat, sid_table, ind_table):
    n = ids.shape[0]
    mesh = plsc.VectorSubcoreMesh(core_axis_name="c", subcore_axis_name="s")
    fn = pl.kernel(
        _sc_body,
        out_type=jax.ShapeDtypeStruct((n, D), jnp.float32),
        mesh=mesh,
        compiler_params=pltpu.CompilerParams(use_tc_tiling_on_sc=False),
        scratch_types=[
            pltpu.VMEM((K,), jnp.int32),
            pltpu.VMEM((L * K,), jnp.int32),
            pltpu.VMEM((K,), jnp.int32),
            pltpu.VMEM((L * K,), jnp.int32),
            pltpu.VMEM((L * K, D), jnp.float32),
            pltpu.VMEM((K, D), jnp.float32),
            pltpu.SemaphoreType.DMA,
            pltpu.SemaphoreType.DMA,
            pltpu.SemaphoreType.DMA,
        ],
    )
    return fn(ids, lookup_flat, sid_table, ind_table)


def kernel(item_ids, lookup, codebook, sid_table, ind_table):
    b, t = item_ids.shape
    ids = item_ids.reshape(-1)
    lookup_flat = lookup.reshape(-1)
    ind_pad = jnp.pad(ind_table, ((0, 0), (0, DP - D)))
    ind_pad = jax.lax.optimization_barrier(ind_pad)
    ind2 = ind_pad.reshape(2 * ind_table.shape[0], D)
    out = _impl(ids, lookup_flat, sid_table, ind2)
    return out.reshape(b, t, D)
